# baseline (speedup 1.0000x reference)
import jax
import jax.numpy as jnp
from jax.experimental import pallas as pl
from jax.experimental.pallas import tpu as pltpu

N = 20000
NPAD = 20480
K = 2000
MAX_DET = 500
NMS_THR = 0.7
SCORE_THR = 0.1
NEG = -1e9


def _prep_kernel(boxes_ref, scores_ref, b_ref, s_ref):
    bx = boxes_ref[...]
    x = bx[:, 0:1] * 1600.0
    y = bx[:, 1:2] * 900.0
    w = bx[:, 2:3] * 120.0 + 4.0
    h = bx[:, 3:4] * 120.0 + 4.0
    b_ref[...] = jnp.concatenate(
        [x - w / 2.0, y - h / 2.0, x + w / 2.0, y + h / 2.0], axis=1)
    s = scores_ref[...]
    s_ref[...] = jnp.where(s > SCORE_THR, s, NEG)


def kernel(boxes, scores):
    boxes_p = jnp.pad(boxes, ((0, NPAD - N), (0, 0)))
    scores_p = jnp.pad(scores, (0, NPAD - N), constant_values=0.0)
    b_p, s_p = pl.pallas_call(
        _prep_kernel,
        out_shape=(
            jax.ShapeDtypeStruct((NPAD, 4), jnp.float32),
            jax.ShapeDtypeStruct((NPAD // 128, 128), jnp.float32),
        ),
    )(boxes_p, scores_p.reshape(NPAD // 128, 128))
    b = b_p[:N]
    s = s_p.reshape(NPAD)[:N]

    top_s, idx = jax.lax.top_k(s, K)
    bb = jnp.take(b, idx, axis=0)
    lt = jnp.maximum(bb[:, None, :2], bb[None, :, :2])
    rb = jnp.minimum(bb[:, None, 2:], bb[None, :, 2:])
    wh = jnp.clip(rb - lt, 0.0, None)
    inter = wh[..., 0] * wh[..., 1]
    area = (bb[:, 2] - bb[:, 0]) * (bb[:, 3] - bb[:, 1])
    union = area[:, None] + area[None, :] - inter
    iou = inter / jnp.maximum(union, 1e-6)
    ar = jnp.arange(K)

    def body(keep, i):
        suppress = (iou[i] > NMS_THR) & (ar > i) & keep[i]
        keep = keep & jnp.logical_not(suppress)
        return keep, None

    keep, _ = jax.lax.scan(body, jnp.ones((K,), dtype=bool), ar)
    final_s = jnp.where(keep & (top_s > NEG / 2), top_s, NEG)
    fs, fidx = jax.lax.top_k(final_s, MAX_DET)
    fb = jnp.take(bb, fidx, axis=0)
    return jnp.concatenate([fb, fs[:, None]], axis=-1)


# speedup vs baseline: 32.3089x; 32.3089x over previous
import jax
import jax.numpy as jnp
from jax.experimental import pallas as pl
from jax.experimental.pallas import tpu as pltpu

N = 20000
NPAD = 20480
K = 2000
MAX_DET = 500
NMS_THR = 0.7
SCORE_THR = 0.1
NEG = -1e9


def _prep_kernel(boxes_ref, scores_ref, b_ref, s_ref):
    bx = boxes_ref[...]
    x = bx[:, 0:1] * 1600.0
    y = bx[:, 1:2] * 900.0
    w = bx[:, 2:3] * 120.0 + 4.0
    h = bx[:, 3:4] * 120.0 + 4.0
    b_ref[...] = jnp.concatenate(
        [x - w / 2.0, y - h / 2.0, x + w / 2.0, y + h / 2.0], axis=1)
    s = scores_ref[...]
    s_ref[...] = jnp.where(s > SCORE_THR, s, NEG)


KPAD = 2048
NBLK = KPAD // 128


def _nms_kernel(bb_ref, bbT_ref, keep_ref, m_ref):
    f32 = jnp.float32
    for rb in range(NBLK):
        rows = slice(rb * 128, (rb + 1) * 128)
        x1r = bb_ref[rows, 0:1]
        y1r = bb_ref[rows, 1:2]
        x2r = bb_ref[rows, 2:3]
        y2r = bb_ref[rows, 3:4]
        area_r = (x2r - x1r) * (y2r - y1r)
        for cb in range(rb, NBLK):
            cols = slice(cb * 128, (cb + 1) * 128)
            x1c = bbT_ref[0:1, cols]
            y1c = bbT_ref[1:2, cols]
            x2c = bbT_ref[2:3, cols]
            y2c = bbT_ref[3:4, cols]
            wx = jnp.maximum(jnp.minimum(x2r, x2c) - jnp.maximum(x1r, x1c), 0.0)
            wy = jnp.maximum(jnp.minimum(y2r, y2c) - jnp.maximum(y1r, y1c), 0.0)
            inter = wx * wy
            area_c = (x2c - x1c) * (y2c - y1c)
            union = area_r + area_c - inter
            iou = inter / jnp.maximum(union, 1e-6)
            m_ref[rows, cols] = (iou > NMS_THR).astype(f32)

    lane = jax.lax.broadcasted_iota(jnp.int32, (1, 128), 1)
    keep_list = [jnp.ones((1, 128), f32) for _ in range(NBLK)]
    for blk in range(NBLK):
        rows = slice(blk * 128, (blk + 1) * 128)
        base = blk * 128

        def body(g, k):
            start = pl.multiple_of(base + g * 8, 8)
            grp = m_ref[pl.ds(start, 8), base:base + 128]
            for r in range(8):
                i = g * 8 + r
                row_i = grp[r:r + 1, :]
                ki = jnp.sum(jnp.where(lane == i, k, 0.0))
                gt = (lane > i).astype(f32)
                k = k * (1.0 - row_i * gt * ki)
            return k

        k = jax.lax.fori_loop(0, 16, body, keep_list[blk])
        keep_list[blk] = k
        for jb in range(blk + 1, NBLK):
            tile = m_ref[rows, jb * 128:(jb + 1) * 128]
            supp = jnp.dot(k, tile, preferred_element_type=f32)
            keep_list[jb] = jnp.where(supp > 0.0, 0.0, keep_list[jb])
    keep_ref[...] = jnp.concatenate(keep_list, axis=0)


def kernel(boxes, scores):
    boxes_p = jnp.pad(boxes, ((0, NPAD - N), (0, 0)))
    scores_p = jnp.pad(scores, (0, NPAD - N), constant_values=0.0)
    b_p, s_p = pl.pallas_call(
        _prep_kernel,
        out_shape=(
            jax.ShapeDtypeStruct((NPAD, 4), jnp.float32),
            jax.ShapeDtypeStruct((NPAD // 128, 128), jnp.float32),
        ),
    )(boxes_p, scores_p.reshape(NPAD // 128, 128))
    b = b_p[:N]
    s = s_p.reshape(NPAD)[:N]

    top_s, idx = jax.lax.top_k(s, K)
    bb = jnp.take(b, idx, axis=0)
    bb_p = jnp.pad(bb, ((0, KPAD - K), (0, 0)))
    keep_f = pl.pallas_call(
        _nms_kernel,
        out_shape=jax.ShapeDtypeStruct((NBLK, 128), jnp.float32),
        scratch_shapes=[pltpu.VMEM((KPAD, KPAD), jnp.float32)],
    )(bb_p, bb_p.T)
    keep = keep_f.reshape(KPAD)[:K] > 0.5

    final_s = jnp.where(keep & (top_s > NEG / 2), top_s, NEG)
    fs, fidx = jax.lax.top_k(final_s, MAX_DET)
    fb = jnp.take(bb, fidx, axis=0)
    return jnp.concatenate([fb, fs[:, None]], axis=-1)


# speedup vs baseline: 117.5973x; 3.6398x over previous
import jax
import jax.numpy as jnp
from jax.experimental import pallas as pl
from jax.experimental.pallas import tpu as pltpu

N = 20000
NPAD = 20480
K = 2000
MAX_DET = 500
NMS_THR = 0.7
SCORE_THR = 0.1
NEG = -1e9


def _prep_kernel(boxes_ref, scores_ref, b_ref, s_ref):
    bx = boxes_ref[...]
    x = bx[:, 0:1] * 1600.0
    y = bx[:, 1:2] * 900.0
    w = bx[:, 2:3] * 120.0 + 4.0
    h = bx[:, 3:4] * 120.0 + 4.0
    b_ref[...] = jnp.concatenate(
        [x - w / 2.0, y - h / 2.0, x + w / 2.0, y + h / 2.0], axis=1)
    s = scores_ref[...]
    s_ref[...] = jnp.where(s > SCORE_THR, s, NEG)


KPAD = 2048
NBLK = KPAD // 128


def _nms_kernel(bb_ref, bbT_ref, keep_ref, m_ref):
    f32 = jnp.float32
    bf16 = jnp.bfloat16
    sub = jax.lax.broadcasted_iota(jnp.int32, (128, 128), 0)
    lane = jax.lax.broadcasted_iota(jnp.int32, (128, 128), 1)
    diag_gt = (lane > sub).astype(bf16)
    for rb in range(NBLK):
        rows = slice(rb * 128, (rb + 1) * 128)
        if rb > 0:
            m_ref[rows, 0:rb * 128] = jnp.zeros((128, rb * 128), bf16)
        x1r = bb_ref[rows, 0:1]
        y1r = bb_ref[rows, 1:2]
        x2r = bb_ref[rows, 2:3]
        y2r = bb_ref[rows, 3:4]
        area_r = (x2r - x1r) * (y2r - y1r)
        for cb in range(rb, NBLK):
            cols = slice(cb * 128, (cb + 1) * 128)
            x1c = bbT_ref[0:1, cols]
            y1c = bbT_ref[1:2, cols]
            x2c = bbT_ref[2:3, cols]
            y2c = bbT_ref[3:4, cols]
            wx = jnp.maximum(jnp.minimum(x2r, x2c) - jnp.maximum(x1r, x1c), 0.0)
            wy = jnp.maximum(jnp.minimum(y2r, y2c) - jnp.maximum(y1r, y1c), 0.0)
            inter = wx * wy
            area_c = (x2c - x1c) * (y2c - y1c)
            union = area_r + area_c - inter
            iou = inter / jnp.maximum(union, 1e-6)
            m = (iou > NMS_THR).astype(bf16)
            if cb == rb:
                m = m * diag_gt
            m_ref[rows, cols] = m

    def cond(carry):
        _, changed = carry
        return changed > 0

    def body(carry):
        k, _ = carry
        parts = []
        for jb in range(NBLK):
            kk = k[:, : (jb + 1) * 128]
            mm = m_ref[0:(jb + 1) * 128, jb * 128:(jb + 1) * 128]
            supp = jnp.dot(kk, mm, preferred_element_type=f32)
            parts.append((supp == 0.0).astype(bf16))
        k_new = jnp.concatenate(parts, axis=1)
        changed = jnp.sum(jnp.abs((k_new - k).astype(f32)))
        return k_new, changed

    k0 = jnp.ones((1, KPAD), bf16)
    k_fin, _ = jax.lax.while_loop(cond, body, (k0, jnp.float32(1.0)))
    for blk in range(NBLK):
        keep_ref[blk:blk + 1, :] = k_fin[:, blk * 128:(blk + 1) * 128].astype(f32)


def kernel(boxes, scores):
    boxes_p = jnp.pad(boxes, ((0, NPAD - N), (0, 0)))
    scores_p = jnp.pad(scores, (0, NPAD - N), constant_values=0.0)
    b_p, s_p = pl.pallas_call(
        _prep_kernel,
        out_shape=(
            jax.ShapeDtypeStruct((NPAD, 4), jnp.float32),
            jax.ShapeDtypeStruct((NPAD // 128, 128), jnp.float32),
        ),
    )(boxes_p, scores_p.reshape(NPAD // 128, 128))
    b = b_p[:N]
    s = s_p.reshape(NPAD)[:N]

    top_s, idx = jax.lax.top_k(s, K)
    bb = jnp.take(b, idx, axis=0)
    bb_p = jnp.pad(bb, ((0, KPAD - K), (0, 0)))
    keep_f = pl.pallas_call(
        _nms_kernel,
        out_shape=jax.ShapeDtypeStruct((NBLK, 128), jnp.float32),
        scratch_shapes=[pltpu.VMEM((KPAD, KPAD), jnp.bfloat16)],
    )(bb_p, bb_p.T)
    keep = keep_f.reshape(KPAD)[:K] > 0.5

    final_s = jnp.where(keep & (top_s > NEG / 2), top_s, NEG)
    fs, fidx = jax.lax.top_k(final_s, MAX_DET)
    fb = jnp.take(bb, fidx, axis=0)
    return jnp.concatenate([fb, fs[:, None]], axis=-1)
